# baseline (speedup 1.0000x reference)
import functools

import jax
import jax.numpy as jnp
from jax import lax
from jax.experimental import pallas as pl
from jax.experimental.pallas import tpu as pltpu
from jax.experimental.pallas import tpu_sc as plsc

N_NODES = 10000
D = 128
E = 320000

NC = 2
NS = 16
NW = NC * NS

CHUNK = 64
NBUF = 3
G = 54
N_CHUNKS = NBUF * G
EDGES_PER_W = CHUNK * N_CHUNKS
E_PAD = NW * EDGES_PER_W
N_ACC = 10240
ROWS_PER_TILE = N_ACC // NS
PAD_DST = N_NODES


def _sc_aggregate(x, rc, zblock):
    mesh = plsc.VectorSubcoreMesh(core_axis_name="c", subcore_axis_name="s")

    @functools.partial(
        pl.kernel,
        mesh=mesh,
        out_type=jax.ShapeDtypeStruct((NC, N_ACC, D), jnp.float32),
        scratch_types=[
            pltpu.VMEM((CHUNK, D), jnp.float32),
            pltpu.VMEM((CHUNK, D), jnp.float32),
            pltpu.VMEM((CHUNK, D), jnp.float32),
            pltpu.VMEM((2 * NBUF, CHUNK), jnp.int32),
            pltpu.VMEM((2 * NBUF, CHUNK), jnp.int32),
            pltpu.VMEM((2 * NBUF, CHUNK), jnp.int32),
            pltpu.VMEM_SHARED((N_ACC, D), jnp.float32),
            pltpu.SemaphoreType.DMA,
            pltpu.SemaphoreType.DMA,
            pltpu.SemaphoreType.DMA,
            pltpu.SemaphoreType.DMA,
            pltpu.SemaphoreType.DMA,
            pltpu.SemaphoreType.DMA,
        ],
    )
    def body(x_hbm, rc_hbm, z_hbm, out_hbm,
             buf0, buf1, buf2, rc0, rc1, rc2, acc_sh,
             sem0, sem1, sem2, isem0, isem1, isem2):
        cid = lax.axis_index("c")
        sid = lax.axis_index("s")
        wid = cid * NS + sid
        bufs = (buf0, buf1, buf2)
        sems = (sem0, sem1, sem2)
        rcbufs = (rc0, rc1, rc2)
        isems = (isem0, isem1, isem2)
        base = wid * G

        pltpu.sync_copy(z_hbm, buf0)
        r0 = sid * ROWS_PER_TILE
        for b in range(ROWS_PER_TILE // CHUNK):
            pltpu.sync_copy(buf0, acc_sh.at[pl.ds(r0 + b * CHUNK, CHUNK)])
        plsc.subcore_barrier()

        pltpu.sync_copy(rc_hbm.at[base], rc0)
        pltpu.async_copy(rc_hbm.at[base + 1], rc1, isem1)
        for b in range(NBUF):
            pltpu.async_copy(x_hbm.at[rc0.at[b]], bufs[b], sems[b])

        def triple(t, carry):
            for p in range(3):
                g = 3 * t + p
                rc_nxt = rcbufs[(p + 1) % 3]
                rc_fre = rcbufs[(p + 2) % 3]

                @pl.when(g + 2 < G)
                def _load_idx(rc_fre=rc_fre, p=p, g=g):
                    pltpu.async_copy(rc_hbm.at[base + g + 2], rc_fre,
                                     isems[(p + 2) % 3])

                @pl.when(g + 1 < G)
                def _wait_idx(rc_nxt=rc_nxt, p=p):
                    pltpu.make_async_copy(rc_hbm.at[base], rc_nxt,
                                          isems[(p + 1) % 3]).wait()

                for b in range(NBUF):
                    pltpu.make_async_copy(x_hbm.at[pl.ds(0, CHUNK)], bufs[b],
                                          sems[b]).wait()
                    pltpu.sync_copy(bufs[b], acc_sh.at[rcbufs[p].at[NBUF + b]],
                                    add=True)

                    @pl.when(g + 1 < G)
                    def _prefetch(rc_nxt=rc_nxt, b=b):
                        pltpu.async_copy(x_hbm.at[rc_nxt.at[b]], bufs[b],
                                         sems[b])
            return carry

        lax.fori_loop(0, G // 3, triple, 0)
        plsc.subcore_barrier()

        pltpu.sync_copy(acc_sh.at[pl.ds(r0, ROWS_PER_TILE)],
                        out_hbm.at[cid, pl.ds(r0, ROWS_PER_TILE)])

    return body(x, rc, zblock)


def _dense_body(p0_ref, p1_ref, x_ref, wl_ref, wr_ref, b_ref, o_ref):
    agg = p0_ref[...] + p1_ref[...]
    o_ref[...] = (
        lax.dot_general(agg, wl_ref[...], (((1,), (1,)), ((), ())),
                        preferred_element_type=jnp.float32)
        + lax.dot_general(x_ref[...], wr_ref[...], (((1,), (1,)), ((), ())),
                          preferred_element_type=jnp.float32)
        + b_ref[...]
    )


def kernel(x, edge_index, W_l, b_l, W_r):
    row = edge_index[0]
    col = edge_index[1]
    npad = E_PAD - E
    row = jnp.concatenate([row, jnp.zeros((npad,), jnp.int32)])
    col = jnp.concatenate([col, jnp.full((npad,), PAD_DST, jnp.int32)])
    row4 = row.reshape(NW, G, NBUF, CHUNK)
    col4 = col.reshape(NW, G, NBUF, CHUNK)
    rc = jnp.concatenate([row4, col4], axis=2).reshape(NW * G, 2 * NBUF, CHUNK)
    zblock = jnp.zeros((CHUNK, D), jnp.float32)

    p = _sc_aggregate(x, rc, zblock)

    blk = 1000
    grid = (N_NODES // blk,)
    out = pl.pallas_call(
        _dense_body,
        grid=grid,
        in_specs=[
            pl.BlockSpec((blk, D), lambda i: (i, 0)),
            pl.BlockSpec((blk, D), lambda i: (i, 0)),
            pl.BlockSpec((blk, D), lambda i: (i, 0)),
            pl.BlockSpec((D, D), lambda i: (0, 0)),
            pl.BlockSpec((D, D), lambda i: (0, 0)),
            pl.BlockSpec((1, D), lambda i: (0, 0)),
        ],
        out_specs=pl.BlockSpec((blk, D), lambda i: (i, 0)),
        out_shape=jax.ShapeDtypeStruct((N_NODES, D), jnp.float32),
    )(p[0], p[1], x, W_l, W_r, b_l.reshape(1, D))
    return out


# speedup vs baseline: 2.4293x; 2.4293x over previous
import functools

import jax
import jax.numpy as jnp
from jax import lax
from jax.experimental import pallas as pl
from jax.experimental.pallas import tpu as pltpu
from jax.experimental.pallas import tpu_sc as plsc

N_NODES = 10000
D = 128
E = 320000

NC = 2
NS = 16
NW = NC * NS

CHUNK = 64
NBUF = 3
G0 = 87
G1 = 18
EDGES_PER_G = NBUF * CHUNK
E_PAD = NS * (G0 + G1) * EDGES_PER_G
N_GRP = NS * (G0 + G1)
N_ACC = 10240
ROWS_PER_TILE = N_ACC // NS
PAD_DST = N_NODES


def _sc_aggregate(x, rc, zblock):
    mesh = plsc.VectorSubcoreMesh(core_axis_name="c", subcore_axis_name="s")

    @functools.partial(
        pl.kernel,
        mesh=mesh,
        out_type=jax.ShapeDtypeStruct((NC, N_ACC, D), jnp.float32),
        scratch_types=[
            pltpu.VMEM((CHUNK, D), jnp.float32),
            pltpu.VMEM((CHUNK, D), jnp.float32),
            pltpu.VMEM((CHUNK, D), jnp.float32),
            pltpu.VMEM((2 * NBUF, CHUNK), jnp.int32),
            pltpu.VMEM((2 * NBUF, CHUNK), jnp.int32),
            pltpu.VMEM((2 * NBUF, CHUNK), jnp.int32),
            pltpu.VMEM_SHARED((N_ACC, D), jnp.float32),
            pltpu.SemaphoreType.DMA,
            pltpu.SemaphoreType.DMA,
            pltpu.SemaphoreType.DMA,
            pltpu.SemaphoreType.DMA,
            pltpu.SemaphoreType.DMA,
            pltpu.SemaphoreType.DMA,
        ],
    )
    def body(x_hbm, rc_hbm, z_hbm, out_hbm,
             buf0, buf1, buf2, rc0, rc1, rc2, acc_sh,
             sem0, sem1, sem2, isem0, isem1, isem2):
        cid = lax.axis_index("c")
        sid = lax.axis_index("s")
        bufs = (buf0, buf1, buf2)
        sems = (sem0, sem1, sem2)
        rcbufs = (rc0, rc1, rc2)
        isems = (isem0, isem1, isem2)

        pltpu.sync_copy(z_hbm, buf0)
        r0 = sid * ROWS_PER_TILE
        for b in range(ROWS_PER_TILE // CHUNK):
            pltpu.sync_copy(buf0, acc_sh.at[pl.ds(r0 + b * CHUNK, CHUNK)])
        plsc.subcore_barrier()

        def run(num_g, base):
            pltpu.sync_copy(rc_hbm.at[base], rc0)
            pltpu.async_copy(rc_hbm.at[base + 1], rc1, isem1)
            for b in range(NBUF):
                pltpu.async_copy(x_hbm.at[rc0.at[b]], bufs[b], sems[b])

            def triple(t, carry):
                for p in range(3):
                    g = 3 * t + p
                    rc_nxt = rcbufs[(p + 1) % 3]
                    rc_fre = rcbufs[(p + 2) % 3]

                    @pl.when(g + 2 < num_g)
                    def _load_idx(rc_fre=rc_fre, p=p, g=g):
                        pltpu.async_copy(rc_hbm.at[base + g + 2], rc_fre,
                                         isems[(p + 2) % 3])

                    @pl.when(g + 1 < num_g)
                    def _wait_idx(rc_nxt=rc_nxt, p=p):
                        pltpu.make_async_copy(rc_hbm.at[base], rc_nxt,
                                              isems[(p + 1) % 3]).wait()

                    for b in range(NBUF):
                        pltpu.make_async_copy(x_hbm.at[pl.ds(0, CHUNK)],
                                              bufs[b], sems[b]).wait()
                        pltpu.sync_copy(bufs[b],
                                        acc_sh.at[rcbufs[p].at[NBUF + b]],
                                        add=True)

                        @pl.when(g + 1 < num_g)
                        def _prefetch(rc_nxt=rc_nxt, b=b):
                            pltpu.async_copy(x_hbm.at[rc_nxt.at[b]], bufs[b],
                                             sems[b])
                return carry

            lax.fori_loop(0, num_g // 3, triple, 0)

        @pl.when(cid == 0)
        def _fast_core():
            run(G0, sid * G0)

        @pl.when(cid == 1)
        def _slow_core():
            run(G1, NS * G0 + sid * G1)

        plsc.subcore_barrier()

        pltpu.sync_copy(acc_sh.at[pl.ds(r0, ROWS_PER_TILE)],
                        out_hbm.at[cid, pl.ds(r0, ROWS_PER_TILE)])

    return body(x, rc, zblock)


def _dense_body(p0_ref, p1_ref, x_ref, wl_ref, wr_ref, b_ref, o_ref):
    agg = p0_ref[...] + p1_ref[...]
    o_ref[...] = (
        lax.dot_general(agg, wl_ref[...], (((1,), (1,)), ((), ())),
                        preferred_element_type=jnp.float32)
        + lax.dot_general(x_ref[...], wr_ref[...], (((1,), (1,)), ((), ())),
                          preferred_element_type=jnp.float32)
        + b_ref[...]
    )


def kernel(x, edge_index, W_l, b_l, W_r):
    row = edge_index[0]
    col = edge_index[1]
    npad = E_PAD - E
    row = jnp.concatenate([row, jnp.zeros((npad,), jnp.int32)])
    col = jnp.concatenate([col, jnp.full((npad,), PAD_DST, jnp.int32)])
    row4 = row.reshape(N_GRP, NBUF, CHUNK)
    col4 = col.reshape(N_GRP, NBUF, CHUNK)
    rc = jnp.concatenate([row4, col4], axis=1)
    zblock = jnp.zeros((CHUNK, D), jnp.float32)

    p = _sc_aggregate(x, rc, zblock)

    blk = 1000
    grid = (N_NODES // blk,)
    out = pl.pallas_call(
        _dense_body,
        grid=grid,
        in_specs=[
            pl.BlockSpec((blk, D), lambda i: (i, 0)),
            pl.BlockSpec((blk, D), lambda i: (i, 0)),
            pl.BlockSpec((blk, D), lambda i: (i, 0)),
            pl.BlockSpec((D, D), lambda i: (0, 0)),
            pl.BlockSpec((D, D), lambda i: (0, 0)),
            pl.BlockSpec((1, D), lambda i: (0, 0)),
        ],
        out_specs=pl.BlockSpec((blk, D), lambda i: (i, 0)),
        out_shape=jax.ShapeDtypeStruct((N_NODES, D), jnp.float32),
    )(p[0], p[1], x, W_l, W_r, b_l.reshape(1, D))
    return out


# speedup vs baseline: 2.6366x; 1.0853x over previous
import functools

import jax
import jax.numpy as jnp
from jax import lax
from jax.experimental import pallas as pl
from jax.experimental.pallas import tpu as pltpu
from jax.experimental.pallas import tpu_sc as plsc

N_NODES = 10000
D = 128
E = 320000

NC = 2
NS = 16
NW = NC * NS

CHUNK = 64
NBUF = 3
G0 = 87
G1 = 18
EDGES_PER_G = NBUF * CHUNK
E_PAD = NS * (G0 + G1) * EDGES_PER_G
N_GRP = NS * (G0 + G1)
N_ACC = 10240
ROWS_PER_TILE = N_ACC // NS
PAD_DST = N_NODES


def _sc_aggregate(xp, ei4, zblock):
    mesh = plsc.VectorSubcoreMesh(core_axis_name="c", subcore_axis_name="s")

    @functools.partial(
        pl.kernel,
        mesh=mesh,
        out_type=jax.ShapeDtypeStruct((NC, N_ACC, D), jnp.float32),
        scratch_types=[
            pltpu.VMEM((CHUNK, D), jnp.float32),
            pltpu.VMEM((CHUNK, D), jnp.float32),
            pltpu.VMEM((CHUNK, D), jnp.float32),
            pltpu.VMEM((NBUF, CHUNK), jnp.int32),
            pltpu.VMEM((NBUF, CHUNK), jnp.int32),
            pltpu.VMEM((NBUF, CHUNK), jnp.int32),
            pltpu.VMEM((NBUF, CHUNK), jnp.int32),
            pltpu.VMEM((NBUF, CHUNK), jnp.int32),
            pltpu.VMEM((NBUF, CHUNK), jnp.int32),
            pltpu.VMEM_SHARED((N_ACC, D), jnp.float32),
            pltpu.SemaphoreType.DMA,
            pltpu.SemaphoreType.DMA,
            pltpu.SemaphoreType.DMA,
            pltpu.SemaphoreType.DMA,
            pltpu.SemaphoreType.DMA,
            pltpu.SemaphoreType.DMA,
        ],
    )
    def body(x_hbm, ei_hbm, z_hbm, out_hbm,
             buf0, buf1, buf2, rr0, rr1, rr2, rc0, rc1, rc2, acc_sh,
             sem0, sem1, sem2, isem0, isem1, isem2):
        cid = lax.axis_index("c")
        sid = lax.axis_index("s")
        bufs = (buf0, buf1, buf2)
        sems = (sem0, sem1, sem2)
        rrow = (rr0, rr1, rr2)
        rcol = (rc0, rc1, rc2)
        isems = (isem0, isem1, isem2)

        pltpu.sync_copy(z_hbm, buf0)
        r0 = sid * ROWS_PER_TILE
        for b in range(ROWS_PER_TILE // CHUNK):
            pltpu.sync_copy(buf0, acc_sh.at[pl.ds(r0 + b * CHUNK, CHUNK)])
        plsc.subcore_barrier()

        def run(num_g, base):
            pltpu.sync_copy(ei_hbm.at[0, base], rrow[0])
            pltpu.sync_copy(ei_hbm.at[1, base], rcol[0])
            pltpu.async_copy(ei_hbm.at[0, base + 1], rrow[1], isem1)
            pltpu.async_copy(ei_hbm.at[1, base + 1], rcol[1], isem1)
            for b in range(NBUF):
                pltpu.async_copy(x_hbm.at[rrow[0].at[b]], bufs[b], sems[b])

            def triple(t, carry):
                for p in range(3):
                    g = 3 * t + p
                    pn = (p + 1) % 3
                    pf = (p + 2) % 3

                    @pl.when(g + 2 < num_g)
                    def _load_idx(pf=pf, g=g):
                        pltpu.async_copy(ei_hbm.at[0, base + g + 2], rrow[pf],
                                         isems[pf])
                        pltpu.async_copy(ei_hbm.at[1, base + g + 2], rcol[pf],
                                         isems[pf])

                    @pl.when(g + 1 < num_g)
                    def _wait_idx(pn=pn):
                        pltpu.make_async_copy(ei_hbm.at[0, base], rrow[pn],
                                              isems[pn]).wait()
                        pltpu.make_async_copy(ei_hbm.at[1, base], rcol[pn],
                                              isems[pn]).wait()

                    for b in range(NBUF):
                        pltpu.make_async_copy(x_hbm.at[pl.ds(0, CHUNK)],
                                              bufs[b], sems[b]).wait()
                        pltpu.sync_copy(bufs[b],
                                        acc_sh.at[rcol[p].at[b]],
                                        add=True)

                        @pl.when(g + 1 < num_g)
                        def _prefetch(pn=pn, b=b):
                            pltpu.async_copy(x_hbm.at[rrow[pn].at[b]], bufs[b],
                                             sems[b])
                return carry

            lax.fori_loop(0, num_g // 3, triple, 0)

        @pl.when(cid == 0)
        def _fast_core():
            run(G0, sid * G0)

        @pl.when(cid == 1)
        def _slow_core():
            run(G1, NS * G0 + sid * G1)

        plsc.subcore_barrier()

        pltpu.sync_copy(acc_sh.at[pl.ds(r0, ROWS_PER_TILE)],
                        out_hbm.at[cid, pl.ds(r0, ROWS_PER_TILE)])

    return body(xp, ei4, zblock)


def _dense_r_body(x_ref, wr_ref, b_ref, o_ref):
    o_ref[...] = lax.dot_general(
        x_ref[...], wr_ref[...], (((1,), (1,)), ((), ())),
        preferred_element_type=jnp.float32) + b_ref[...]


def _dense_l_body(p0_ref, p1_ref, yr_ref, wl_ref, o_ref):
    agg = p0_ref[0] + p1_ref[0]
    o_ref[...] = lax.dot_general(
        agg, wl_ref[...], (((1,), (1,)), ((), ())),
        preferred_element_type=jnp.float32) + yr_ref[...]


def kernel(x, edge_index, W_l, b_l, W_r):
    npad = E_PAD - E
    ei4 = jnp.pad(edge_index, ((0, 0), (0, npad)),
                  constant_values=PAD_DST).reshape(2, N_GRP, NBUF, CHUNK)
    xp = jnp.pad(x, ((0, N_ACC - N_NODES), (0, 0)))
    zblock = jnp.zeros((CHUNK, D), jnp.float32)

    blk = 1000
    grid = (N_NODES // blk,)

    y_r = pl.pallas_call(
        _dense_r_body,
        grid=grid,
        in_specs=[
            pl.BlockSpec((blk, D), lambda i: (i, 0)),
            pl.BlockSpec((D, D), lambda i: (0, 0)),
            pl.BlockSpec((1, D), lambda i: (0, 0)),
        ],
        out_specs=pl.BlockSpec((blk, D), lambda i: (i, 0)),
        out_shape=jax.ShapeDtypeStruct((N_NODES, D), jnp.float32),
    )(x, W_r, b_l.reshape(1, D))

    p = _sc_aggregate(xp, ei4, zblock)

    out = pl.pallas_call(
        _dense_l_body,
        grid=grid,
        in_specs=[
            pl.BlockSpec((1, blk, D), lambda i: (0, i, 0)),
            pl.BlockSpec((1, blk, D), lambda i: (1, i, 0)),
            pl.BlockSpec((blk, D), lambda i: (i, 0)),
            pl.BlockSpec((D, D), lambda i: (0, 0)),
        ],
        out_specs=pl.BlockSpec((blk, D), lambda i: (i, 0)),
        out_shape=jax.ShapeDtypeStruct((N_NODES, D), jnp.float32),
    )(p, p, y_r, W_l)
    return out
